# baseline (speedup 1.0000x reference)
import functools

import jax
import jax.numpy as jnp
from jax import lax
from jax.experimental import pallas as pl
from jax.experimental.pallas import tpu as pltpu
from jax.experimental.pallas import tpu_sc as plsc

N = 10000
E = 160000
D = 256
HID = 256
NCLS = 40

NP = 10240
TRASH = 10240
ACC = 10368
NTILES = 16
EPT = E // NTILES
CH = 80
RPT = NP // NTILES

NPH = NP // 2
TRASH2 = NPH
ACC2 = NPH + 128
RPT2 = NPH // NTILES

_HIGH = jax.lax.Precision.HIGHEST


def _dot(a, b):
    return jax.lax.dot(a, b, precision=_HIGH, preferred_element_type=jnp.float32)


def _sc_mesh():
    return plsc.VectorSubcoreMesh(core_axis_name="c", subcore_axis_name="s",
                                  num_cores=2, num_subcores=16)


def _make_degree_kernel():
    @functools.partial(
        pl.kernel,
        out_type=jax.ShapeDtypeStruct((2, NP), jnp.float32),
        mesh=_sc_mesh(),
        scratch_types=[
            pltpu.VMEM((40, 128), jnp.int32),
            pltpu.VMEM((128,), jnp.float32),
            pltpu.VMEM((RPT,), jnp.float32),
            pltpu.VMEM_SHARED((ACC,), jnp.float32),
            pltpu.SemaphoreType.DMA,
            pltpu.SemaphoreType.DMA,
        ],
    )
    def deg_kernel(dstidx_hbm, out_hbm, didx, ones, zb, acc, semA, semB):
        c = lax.axis_index("c")
        s = lax.axis_index("s")
        pltpu.sync_copy(dstidx_hbm.at[s, pl.ds(c * 40, 40)], didx)

        @pl.loop(0, 8)
        def _(r):
            ones[pl.ds(r * 16, 16)] = jnp.full((16,), 1.0, jnp.float32)

        @pl.loop(0, RPT // 16)
        def _(r):
            zb[pl.ds(r * 16, 16)] = jnp.zeros((16,), jnp.float32)

        pltpu.sync_copy(zb, acc.at[pl.ds(s * RPT, RPT)])
        plsc.subcore_barrier()

        def issue(j, sem):
            pltpu.async_copy(ones, acc.at[didx.at[j]], sem, add=True)

        def wait(sem):
            pltpu.make_async_copy(ones, acc.at[pl.ds(0, 128)], sem).wait()

        issue(0, semA)
        issue(1, semB)

        @pl.loop(0, 19)
        def _(t):
            wait(semA)
            issue(2 * t + 2, semA)
            wait(semB)
            issue(2 * t + 3, semB)

        wait(semA)
        wait(semB)
        plsc.subcore_barrier()
        pltpu.sync_copy(acc.at[pl.ds(s * RPT, RPT)],
                        out_hbm.at[c, pl.ds(s * RPT, RPT)])

    return deg_kernel


def _make_propagate_kernel(n_phases, node_div):
    nph = NP // node_div
    rpt = nph // NTILES
    accr = nph + 64

    @functools.partial(
        pl.kernel,
        out_type=jax.ShapeDtypeStruct((n_phases, NP, 128), jnp.float32),
        mesh=_sc_mesh(),
        scratch_types=[
            pltpu.VMEM((CH, 128), jnp.int32),
            pltpu.VMEM((CH, 128), jnp.int32),
            pltpu.VMEM((128, 128), jnp.float32),
            pltpu.VMEM((128, 128), jnp.float32),
            pltpu.VMEM((128, 128), jnp.float32),
            pltpu.VMEM((128, 128), jnp.float32),
            pltpu.SemaphoreType.DMA,
            pltpu.SemaphoreType.DMA,
            pltpu.SemaphoreType.DMA,
            pltpu.SemaphoreType.DMA,
            pltpu.VMEM_SHARED((accr, 128), jnp.float32),
        ],
    )
    def prop_kernel(u_hbm, srcidx_hbm, dstidx_hbm, out_hbm,
                    sidx, didx, b00, b01, b10, b11,
                    gs0, gs1, ss0, ss1, acc):
        c = lax.axis_index("c")
        s = lax.axis_index("s")

        bufs = ((b00, b01), (b10, b11))
        gsem = (gs0, gs1)
        ssem = (ss0, ss1)
        NG = CH // 2

        def issue_gathers(g, p, k):
            for b in range(2):
                pltpu.async_copy(u_hbm.at[k].at[sidx.at[g * 2 + b]],
                                 bufs[p][b], gsem[p])

        def wait_gathers(p):
            for b in range(2):
                pltpu.make_async_copy(u_hbm.at[0].at[pl.ds(0, 128)],
                                      bufs[p][b], gsem[p]).wait()

        def issue_scatters(g, p):
            for b in range(2):
                pltpu.async_copy(bufs[p][b], acc.at[didx.at[g * 2 + b]],
                                 ssem[p], add=True)

        def wait_scatters(p):
            for b in range(2):
                pltpu.make_async_copy(bufs[p][b], acc.at[pl.ds(0, 128)],
                                      ssem[p]).wait()

        pltpu.sync_copy(srcidx_hbm.at[s], sidx)
        for k in range(n_phases):
            for sp in range(node_div // 2):
                pltpu.sync_copy(dstidx_hbm.at[sp, c, s], didx)

                @pl.loop(0, 128)
                def _(r):
                    for q in range(8):
                        b00[r, pl.ds(q * 16, 16)] = (
                            jnp.zeros((16,), jnp.float32))
                nfull, rem = divmod(rpt, 128)
                for z in range(nfull):
                    pltpu.sync_copy(
                        b00, acc.at[pl.ds(s * rpt + z * 128, 128)])
                if rem:
                    pltpu.sync_copy(
                        b00.at[pl.ds(0, rem)],
                        acc.at[pl.ds(s * rpt + nfull * 128, rem)])

                plsc.subcore_barrier()

                def grp(g, p):
                    wait_gathers(p)
                    issue_scatters(g, p)
                    wait_scatters(1 - p)
                    issue_gathers(g + 1, 1 - p, k)

                issue_gathers(0, 0, k)
                wait_gathers(0)
                issue_scatters(0, 0)
                issue_gathers(1, 1, k)

                @pl.loop(0, (NG - 2) // 2)
                def _(t):
                    grp(2 * t + 1, 1)
                    grp(2 * t + 2, 0)

                wait_gathers(1)
                issue_scatters(NG - 1, 1)
                wait_scatters(0)
                wait_scatters(1)
                plsc.subcore_barrier()
                pltpu.sync_copy(
                    acc.at[pl.ds(s * rpt, rpt)],
                    out_hbm.at[k, pl.ds((2 * sp + c) * nph + s * rpt, rpt)])

    return prop_kernel


@functools.lru_cache(maxsize=None)
def _get_degree_kernel():
    return _make_degree_kernel()


@functools.lru_cache(maxsize=None)
def _get_propagate_kernel(n_phases, node_div):
    return _make_propagate_kernel(n_phases, node_div)


def _degree(dstidx):
    return _get_degree_kernel()(dstidx)


def _prop_hidden(u, srcidx, dstidx_r, node_div):
    return _get_propagate_kernel(2, node_div)(u, srcidx, dstidx_r)


def _prop_out(u3p, srcidx, dstidx2):
    return _get_propagate_kernel(1, 2)(u3p.reshape(1, NP, 128), srcidx,
                                       dstidx2).reshape(NP, 128)


BR = 256
NB = NP // BR


def _dinv(deg_ref):
    d = deg_ref[:, 0:1] + deg_ref[:, 1:2] + 1.0
    return jax.lax.rsqrt(d)


def _cat2(u_ref):
    return jnp.concatenate([u_ref[0], u_ref[1]], axis=1)


def _halves(z):
    return jnp.stack([z[:, 0:128], z[:, 128:256]])


def _mm1_body(x_ref, w_ref, deg_ref, u_ref):
    dinv = _dinv(deg_ref)
    z = _dot(x_ref[...], w_ref[...]) * dinv
    u_ref[...] = _halves(z)


def _mm1(x_pad, W1, degT):
    return pl.pallas_call(
        _mm1_body,
        grid=(NB,),
        in_specs=[
            pl.BlockSpec((BR, D), lambda i: (i, 0)),
            pl.BlockSpec((D, HID), lambda i: (0, 0)),
            pl.BlockSpec((BR, 2), lambda i: (i, 0)),
        ],
        out_specs=pl.BlockSpec((2, BR, 128), lambda i: (0, i, 0)),
        out_shape=jax.ShapeDtypeStruct((2, NP, 128), jnp.float32),
    )(x_pad, W1, degT)


def _mm2_body(s_ref, u_ref, deg_ref, b_ref, w_ref, u2_ref, h_ref):
    dinv = _dinv(deg_ref)
    h1 = jnp.maximum(dinv * (_cat2(s_ref) + _cat2(u_ref)) + b_ref[...], 0.0)
    h_ref[...] = h1
    z = _dot(h1, w_ref[...]) * dinv
    u2_ref[...] = _halves(z)


def _mm2(s1, u1, degT, b2d, W):
    return pl.pallas_call(
        _mm2_body,
        grid=(NB,),
        in_specs=[
            pl.BlockSpec((2, BR, 128), lambda i: (0, i, 0)),
            pl.BlockSpec((2, BR, 128), lambda i: (0, i, 0)),
            pl.BlockSpec((BR, 2), lambda i: (i, 0)),
            pl.BlockSpec((1, HID), lambda i: (0, 0)),
            pl.BlockSpec((HID, HID), lambda i: (0, 0)),
        ],
        out_specs=[
            pl.BlockSpec((2, BR, 128), lambda i: (0, i, 0)),
            pl.BlockSpec((BR, HID), lambda i: (i, 0)),
        ],
        out_shape=[
            jax.ShapeDtypeStruct((2, NP, 128), jnp.float32),
            jax.ShapeDtypeStruct((NP, HID), jnp.float32),
        ],
    )(s1, u1, degT, b2d, W)


def _mm3_body(x_ref, h1_ref, s_ref, u_ref, deg_ref, b_ref, w_ref, u3_ref):
    dinv = _dinv(deg_ref)
    h2 = jnp.maximum(dinv * (_cat2(s_ref) + _cat2(u_ref)) + b_ref[...], 0.0)
    z = (_dot(x_ref[...], w_ref[0:D, :])
         + _dot(h1_ref[...], w_ref[D:D + HID, :])
         + _dot(h2, w_ref[D + HID:, :]))
    u = z * dinv
    u3_ref[...] = jnp.concatenate(
        [u, jnp.zeros((BR, 64), jnp.float32)], axis=1)


def _mm3(x_pad, h1, s2, u2, degT, b2d, Wo_pad):
    return pl.pallas_call(
        _mm3_body,
        grid=(NB,),
        in_specs=[
            pl.BlockSpec((BR, D), lambda i: (i, 0)),
            pl.BlockSpec((BR, HID), lambda i: (i, 0)),
            pl.BlockSpec((2, BR, 128), lambda i: (0, i, 0)),
            pl.BlockSpec((2, BR, 128), lambda i: (0, i, 0)),
            pl.BlockSpec((BR, 2), lambda i: (i, 0)),
            pl.BlockSpec((1, HID), lambda i: (0, 0)),
            pl.BlockSpec((D + 2 * HID, 64), lambda i: (0, 0)),
        ],
        out_specs=pl.BlockSpec((BR, 128), lambda i: (i, 0)),
        out_shape=jax.ShapeDtypeStruct((NP, 128), jnp.float32),
    )(x_pad, h1, s2, u2, degT, b2d, Wo_pad)


def _mm4_body(s_ref, u_ref, deg_ref, b_ref, o_ref):
    dinv = _dinv(deg_ref)
    cat = s_ref[:, 0:64] + u_ref[:, 0:64]
    o_ref[...] = jnp.maximum(dinv * cat + b_ref[...], 0.0)


def _mm4(s3, u3p, degT, bo2d):
    return pl.pallas_call(
        _mm4_body,
        grid=(NB,),
        in_specs=[
            pl.BlockSpec((BR, 128), lambda i: (i, 0)),
            pl.BlockSpec((BR, 128), lambda i: (i, 0)),
            pl.BlockSpec((BR, 2), lambda i: (i, 0)),
            pl.BlockSpec((1, 64), lambda i: (0, 0)),
        ],
        out_specs=pl.BlockSpec((BR, 64), lambda i: (i, 0)),
        out_shape=jax.ShapeDtypeStruct((NP, 64), jnp.float32),
    )(s3, u3p, degT, bo2d)


def kernel(x, edge_index, W1, b1, W2, b2, W_out, b_out):
    src = edge_index[0].astype(jnp.int32)
    dst = edge_index[1].astype(jnp.int32)

    src16 = src.reshape(NTILES, EPT)
    dst16 = dst.reshape(NTILES, EPT)
    pad_e = CH * 128 - EPT
    src_l = jnp.concatenate(
        [src16, jnp.zeros((NTILES, pad_e), jnp.int32)], axis=1
    ).reshape(NTILES, CH, 128)
    srcidx = src_l
    dstidx = jnp.concatenate(
        [dst16, jnp.full((NTILES, pad_e), TRASH, jnp.int32)], axis=1
    ).reshape(NTILES, CH, 128)
    dst_s = jnp.concatenate(
        [dst16, jnp.full((NTILES, pad_e), -1, jnp.int32)], axis=1
    ).reshape(NTILES, CH, 128)

    trash_spread = (jnp.arange(128, dtype=jnp.int32) % 64)[None, None, :]

    def route(node_div):
        nph = NP // node_div
        return jnp.stack([
            jnp.stack([
                jnp.where((dst_s >= q * nph) & (dst_s < (q + 1) * nph),
                          dst_s - q * nph, nph + trash_spread)
                for q in (2 * p, 2 * p + 1)], axis=0)
            for p in range(node_div // 2)])

    dstidx2 = route(2)

    x_pad = jnp.pad(x, ((0, NP - N), (0, 0)))
    Wo_pad = jnp.pad(W_out, ((0, 0), (0, 64 - NCLS)))
    bo_pad = jnp.pad(b_out, (0, 64 - NCLS)).reshape(1, 64)
    b1_2d = b1.reshape(1, HID)
    b2_2d = b2.reshape(1, HID)

    deg2 = _degree(dstidx)
    degT = deg2.T

    u1 = _mm1(x_pad, W1, degT)
    s1 = _prop_hidden(u1, srcidx, dstidx2, 2)
    u2, h1 = _mm2(s1, u1, degT, b1_2d, W2)
    s2 = _prop_hidden(u2, srcidx, dstidx2, 2)
    u3p = _mm3(x_pad, h1, s2, u2, degT, b2_2d, Wo_pad)
    s3 = _prop_out(u3p, srcidx, dstidx2)
    out = _mm4(s3, u3p, degT, bo_pad)
    return out[:N, :NCLS]


# speedup vs baseline: 1.0018x; 1.0018x over previous
import functools

import jax
import jax.numpy as jnp
from jax import lax
from jax.experimental import pallas as pl
from jax.experimental.pallas import tpu as pltpu
from jax.experimental.pallas import tpu_sc as plsc

N = 10000
E = 160000
D = 256
HID = 256
NCLS = 40

NP = 10240
TRASH = 10240
ACC = 10368
NTILES = 16
EPT = E // NTILES
CH = 80
RPT = NP // NTILES

NPH = NP // 2

_HIGH = jax.lax.Precision.HIGHEST


def _dot(a, b):
    return jax.lax.dot(a, b, precision=_HIGH, preferred_element_type=jnp.float32)


def _sc_mesh():
    return plsc.VectorSubcoreMesh(core_axis_name="c", subcore_axis_name="s",
                                  num_cores=2, num_subcores=16)


def _make_degree_kernel():
    @functools.partial(
        pl.kernel,
        out_type=jax.ShapeDtypeStruct((2, NP), jnp.float32),
        mesh=_sc_mesh(),
        scratch_types=[
            pltpu.VMEM((40, 128), jnp.int32),
            pltpu.VMEM((128,), jnp.float32),
            pltpu.VMEM((RPT,), jnp.float32),
            pltpu.VMEM_SHARED((ACC,), jnp.float32),
            pltpu.SemaphoreType.DMA,
            pltpu.SemaphoreType.DMA,
        ],
    )
    def deg_kernel(dstidx_hbm, out_hbm, didx, ones, zb, acc, semA, semB):
        c = lax.axis_index("c")
        s = lax.axis_index("s")
        pltpu.sync_copy(dstidx_hbm.at[s, pl.ds(c * 40, 40)], didx)

        @pl.loop(0, 8)
        def _(r):
            ones[pl.ds(r * 16, 16)] = jnp.full((16,), 1.0, jnp.float32)

        @pl.loop(0, RPT // 16)
        def _(r):
            zb[pl.ds(r * 16, 16)] = jnp.zeros((16,), jnp.float32)

        pltpu.sync_copy(zb, acc.at[pl.ds(s * RPT, RPT)])
        plsc.subcore_barrier()

        def issue(j, sem):
            pltpu.async_copy(ones, acc.at[didx.at[j]], sem, add=True)

        def wait(sem):
            pltpu.make_async_copy(ones, acc.at[pl.ds(0, 128)], sem).wait()

        issue(0, semA)
        issue(1, semB)

        @pl.loop(0, 19)
        def _(t):
            wait(semA)
            issue(2 * t + 2, semA)
            wait(semB)
            issue(2 * t + 3, semB)

        wait(semA)
        wait(semB)
        plsc.subcore_barrier()
        pltpu.sync_copy(acc.at[pl.ds(s * RPT, RPT)],
                        out_hbm.at[c, pl.ds(s * RPT, RPT)])

    return deg_kernel


def _make_propagate_kernel(n_phases, node_div):
    nph = NP // node_div
    rpt = nph // NTILES
    accr = nph + 64

    @functools.partial(
        pl.kernel,
        out_type=jax.ShapeDtypeStruct((n_phases, NP, 128), jnp.float32),
        mesh=_sc_mesh(),
        scratch_types=[
            pltpu.VMEM((CH, 128), jnp.int32),
            pltpu.VMEM((CH, 128), jnp.int32),
            pltpu.VMEM((128, 128), jnp.float32),
            pltpu.VMEM((128, 128), jnp.float32),
            pltpu.VMEM((128, 128), jnp.float32),
            pltpu.VMEM((128, 128), jnp.float32),
            pltpu.SemaphoreType.DMA,
            pltpu.SemaphoreType.DMA,
            pltpu.SemaphoreType.DMA,
            pltpu.SemaphoreType.DMA,
            pltpu.VMEM_SHARED((accr, 128), jnp.float32),
        ],
    )
    def prop_kernel(u_hbm, srcidx_hbm, dstidx_hbm, out_hbm,
                    sidx, didx, b00, b01, b10, b11,
                    gs0, gs1, ss0, ss1, acc):
        c = lax.axis_index("c")
        s = lax.axis_index("s")

        bufs = ((b00, b01), (b10, b11))
        gsem = (gs0, gs1)
        ssem = (ss0, ss1)
        NG = CH // 2

        def issue_gathers(g, p, k):
            for b in range(2):
                pltpu.async_copy(u_hbm.at[k].at[sidx.at[g * 2 + b]],
                                 bufs[p][b], gsem[p])

        def wait_gathers(p):
            for b in range(2):
                pltpu.make_async_copy(u_hbm.at[0].at[pl.ds(0, 128)],
                                      bufs[p][b], gsem[p]).wait()

        def issue_scatters(g, p):
            for b in range(2):
                pltpu.async_copy(bufs[p][b], acc.at[didx.at[g * 2 + b]],
                                 ssem[p], add=True)

        def wait_scatters(p):
            for b in range(2):
                pltpu.make_async_copy(bufs[p][b], acc.at[pl.ds(0, 128)],
                                      ssem[p]).wait()

        pltpu.sync_copy(srcidx_hbm.at[s], sidx)
        for k in range(n_phases):
            for sp in range(node_div // 2):
                pltpu.sync_copy(dstidx_hbm.at[sp, c, s], didx)

                @pl.loop(0, 128)
                def _(r):
                    for q in range(8):
                        b00[r, pl.ds(q * 16, 16)] = (
                            jnp.zeros((16,), jnp.float32))
                nfull, rem = divmod(rpt, 128)
                for z in range(nfull):
                    pltpu.sync_copy(
                        b00, acc.at[pl.ds(s * rpt + z * 128, 128)])
                if rem:
                    pltpu.sync_copy(
                        b00.at[pl.ds(0, rem)],
                        acc.at[pl.ds(s * rpt + nfull * 128, rem)])

                plsc.subcore_barrier()

                def grp(g, p):
                    wait_gathers(p)
                    issue_scatters(g, p)
                    wait_scatters(1 - p)
                    issue_gathers(g + 1, 1 - p, k)

                issue_gathers(0, 0, k)
                wait_gathers(0)
                issue_scatters(0, 0)
                issue_gathers(1, 1, k)

                @pl.loop(0, (NG - 2) // 2)
                def _(t):
                    grp(2 * t + 1, 1)
                    grp(2 * t + 2, 0)

                wait_gathers(1)
                issue_scatters(NG - 1, 1)
                wait_scatters(0)
                wait_scatters(1)
                plsc.subcore_barrier()
                pltpu.sync_copy(
                    acc.at[pl.ds(s * rpt, rpt)],
                    out_hbm.at[k, pl.ds((2 * sp + c) * nph + s * rpt, rpt)])

    return prop_kernel


@functools.lru_cache(maxsize=None)
def _get_degree_kernel():
    return _make_degree_kernel()


@functools.lru_cache(maxsize=None)
def _get_propagate_kernel(n_phases, node_div):
    return _make_propagate_kernel(n_phases, node_div)


def _degree(dstidx):
    return _get_degree_kernel()(dstidx)


def _prop_hidden(u, srcidx, dstidx_r, node_div):
    return _get_propagate_kernel(2, node_div)(u, srcidx, dstidx_r)


def _prop_out(u3p, srcidx, dstidx2):
    return _get_propagate_kernel(1, 2)(u3p.reshape(1, NP, 128), srcidx,
                                       dstidx2).reshape(NP, 128)


BR = 256
NB = NP // BR


def _dinv(deg_ref):
    d = deg_ref[:, 0:1] + deg_ref[:, 1:2] + 1.0
    return jax.lax.rsqrt(d)


def _cat2(u_ref):
    return jnp.concatenate([u_ref[0], u_ref[1]], axis=1)


def _halves(z):
    return jnp.stack([z[:, 0:128], z[:, 128:256]])


def _mm1_body(x_ref, w_ref, deg_ref, u_ref):
    dinv = _dinv(deg_ref)
    z = _dot(x_ref[...], w_ref[...]) * dinv
    u_ref[...] = _halves(z)


def _mm1(x_pad, W1, degT):
    return pl.pallas_call(
        _mm1_body,
        grid=(NB,),
        in_specs=[
            pl.BlockSpec((BR, D), lambda i: (i, 0)),
            pl.BlockSpec((D, HID), lambda i: (0, 0)),
            pl.BlockSpec((BR, 2), lambda i: (i, 0)),
        ],
        out_specs=pl.BlockSpec((2, BR, 128), lambda i: (0, i, 0)),
        out_shape=jax.ShapeDtypeStruct((2, NP, 128), jnp.float32),
    )(x_pad, W1, degT)


def _mm2_body(s_ref, u_ref, deg_ref, b_ref, w_ref, u2_ref, h_ref):
    dinv = _dinv(deg_ref)
    h1 = jnp.maximum(dinv * (_cat2(s_ref) + _cat2(u_ref)) + b_ref[...], 0.0)
    h_ref[...] = h1
    z = _dot(h1, w_ref[...]) * dinv
    u2_ref[...] = _halves(z)


def _mm2(s1, u1, degT, b2d, W):
    return pl.pallas_call(
        _mm2_body,
        grid=(NB,),
        in_specs=[
            pl.BlockSpec((2, BR, 128), lambda i: (0, i, 0)),
            pl.BlockSpec((2, BR, 128), lambda i: (0, i, 0)),
            pl.BlockSpec((BR, 2), lambda i: (i, 0)),
            pl.BlockSpec((1, HID), lambda i: (0, 0)),
            pl.BlockSpec((HID, HID), lambda i: (0, 0)),
        ],
        out_specs=[
            pl.BlockSpec((2, BR, 128), lambda i: (0, i, 0)),
            pl.BlockSpec((BR, HID), lambda i: (i, 0)),
        ],
        out_shape=[
            jax.ShapeDtypeStruct((2, NP, 128), jnp.float32),
            jax.ShapeDtypeStruct((NP, HID), jnp.float32),
        ],
    )(s1, u1, degT, b2d, W)


def _mm3_body(x_ref, h1_ref, s_ref, u_ref, deg_ref, b_ref, w_ref, u3_ref):
    dinv = _dinv(deg_ref)
    h2 = jnp.maximum(dinv * (_cat2(s_ref) + _cat2(u_ref)) + b_ref[...], 0.0)
    z = (_dot(x_ref[...], w_ref[0:D, :])
         + _dot(h1_ref[...], w_ref[D:D + HID, :])
         + _dot(h2, w_ref[D + HID:, :]))
    u = z * dinv
    u3_ref[...] = jnp.concatenate(
        [u, jnp.zeros((BR, 64), jnp.float32)], axis=1)


def _mm3(x_pad, h1, s2, u2, degT, b2d, Wo_pad):
    return pl.pallas_call(
        _mm3_body,
        grid=(NB,),
        in_specs=[
            pl.BlockSpec((BR, D), lambda i: (i, 0)),
            pl.BlockSpec((BR, HID), lambda i: (i, 0)),
            pl.BlockSpec((2, BR, 128), lambda i: (0, i, 0)),
            pl.BlockSpec((2, BR, 128), lambda i: (0, i, 0)),
            pl.BlockSpec((BR, 2), lambda i: (i, 0)),
            pl.BlockSpec((1, HID), lambda i: (0, 0)),
            pl.BlockSpec((D + 2 * HID, 64), lambda i: (0, 0)),
        ],
        out_specs=pl.BlockSpec((BR, 128), lambda i: (i, 0)),
        out_shape=jax.ShapeDtypeStruct((NP, 128), jnp.float32),
    )(x_pad, h1, s2, u2, degT, b2d, Wo_pad)


def _mm4_body(s_ref, u_ref, deg_ref, b_ref, o_ref):
    dinv = _dinv(deg_ref)
    cat = s_ref[:, 0:64] + u_ref[:, 0:64]
    o_ref[...] = jnp.maximum(dinv * cat + b_ref[...], 0.0)


def _mm4(s3, u3p, degT, bo2d):
    return pl.pallas_call(
        _mm4_body,
        grid=(NB,),
        in_specs=[
            pl.BlockSpec((BR, 128), lambda i: (i, 0)),
            pl.BlockSpec((BR, 128), lambda i: (i, 0)),
            pl.BlockSpec((BR, 2), lambda i: (i, 0)),
            pl.BlockSpec((1, 64), lambda i: (0, 0)),
        ],
        out_specs=pl.BlockSpec((BR, 64), lambda i: (i, 0)),
        out_shape=jax.ShapeDtypeStruct((NP, 64), jnp.float32),
    )(s3, u3p, degT, bo2d)


def kernel(x, edge_index, W1, b1, W2, b2, W_out, b_out):
    src = edge_index[0].astype(jnp.int32)
    dst = edge_index[1].astype(jnp.int32)

    src16 = src.reshape(NTILES, EPT)
    dst16 = dst.reshape(NTILES, EPT)
    pad_e = CH * 128 - EPT
    src_l = jnp.concatenate(
        [src16, jnp.zeros((NTILES, pad_e), jnp.int32)], axis=1
    ).reshape(NTILES, CH, 128)
    srcidx = src_l
    dstidx = jnp.concatenate(
        [dst16, jnp.full((NTILES, pad_e), TRASH, jnp.int32)], axis=1
    ).reshape(NTILES, CH, 128)
    dst_s = jnp.concatenate(
        [dst16, jnp.full((NTILES, pad_e), -1, jnp.int32)], axis=1
    ).reshape(NTILES, CH, 128)

    trash_spread = (jnp.arange(128, dtype=jnp.int32) % 64)[None, None, :]

    def route(node_div):
        nph = NP // node_div
        return jnp.stack([
            jnp.stack([
                jnp.where((dst_s >= q * nph) & (dst_s < (q + 1) * nph),
                          dst_s - q * nph, nph + trash_spread)
                for q in (2 * p, 2 * p + 1)], axis=0)
            for p in range(node_div // 2)])

    dstidx2 = route(2)

    x_pad = jnp.pad(x, ((0, NP - N), (0, 0)))
    Wo_pad = jnp.pad(W_out, ((0, 0), (0, 64 - NCLS)))
    bo_pad = jnp.pad(b_out, (0, 64 - NCLS)).reshape(1, 64)
    b1_2d = b1.reshape(1, HID)
    b2_2d = b2.reshape(1, HID)

    deg2 = _degree(dstidx)
    degT = deg2.T

    u1 = _mm1(x_pad, W1, degT)
    s1 = _prop_hidden(u1, srcidx, dstidx2, 2)
    u2, h1 = _mm2(s1, u1, degT, b1_2d, W2)
    s2 = _prop_hidden(u2, srcidx, dstidx2, 2)
    u3p = _mm3(x_pad, h1, s2, u2, degT, b2_2d, Wo_pad)
    s3 = _prop_out(u3p, srcidx, dstidx2)
    out = _mm4(s3, u3p, degT, bo_pad)
    return out[:N, :NCLS]


# speedup vs baseline: 1.0095x; 1.0076x over previous
import functools

import jax
import jax.numpy as jnp
from jax import lax
from jax.experimental import pallas as pl
from jax.experimental.pallas import tpu as pltpu
from jax.experimental.pallas import tpu_sc as plsc

N = 10000
E = 160000
D = 256
HID = 256
NCLS = 40

NP = 10240
TRASH = 10240
ACC = 10368
NTILES = 16
EPT = E // NTILES
CH = 80
RPT = NP // NTILES

NPH = NP // 2

def _dot(a, b):
    return jax.lax.dot(a, b, preferred_element_type=jnp.float32)


def _sc_mesh():
    return plsc.VectorSubcoreMesh(core_axis_name="c", subcore_axis_name="s",
                                  num_cores=2, num_subcores=16)


def _make_degree_kernel():
    @functools.partial(
        pl.kernel,
        out_type=jax.ShapeDtypeStruct((2, NP), jnp.float32),
        mesh=_sc_mesh(),
        scratch_types=[
            pltpu.VMEM((40, 128), jnp.int32),
            pltpu.VMEM((128,), jnp.float32),
            pltpu.VMEM((RPT,), jnp.float32),
            pltpu.VMEM_SHARED((ACC,), jnp.float32),
            pltpu.SemaphoreType.DMA,
            pltpu.SemaphoreType.DMA,
        ],
    )
    def deg_kernel(dstidx_hbm, out_hbm, didx, ones, zb, acc, semA, semB):
        c = lax.axis_index("c")
        s = lax.axis_index("s")
        pltpu.sync_copy(dstidx_hbm.at[s, pl.ds(c * 40, 40)], didx)

        @pl.loop(0, 8)
        def _(r):
            ones[pl.ds(r * 16, 16)] = jnp.full((16,), 1.0, jnp.float32)

        @pl.loop(0, RPT // 16)
        def _(r):
            zb[pl.ds(r * 16, 16)] = jnp.zeros((16,), jnp.float32)

        pltpu.sync_copy(zb, acc.at[pl.ds(s * RPT, RPT)])
        plsc.subcore_barrier()

        def issue(j, sem):
            pltpu.async_copy(ones, acc.at[didx.at[j]], sem, add=True)

        def wait(sem):
            pltpu.make_async_copy(ones, acc.at[pl.ds(0, 128)], sem).wait()

        issue(0, semA)
        issue(1, semB)

        @pl.loop(0, 19)
        def _(t):
            wait(semA)
            issue(2 * t + 2, semA)
            wait(semB)
            issue(2 * t + 3, semB)

        wait(semA)
        wait(semB)
        plsc.subcore_barrier()
        pltpu.sync_copy(acc.at[pl.ds(s * RPT, RPT)],
                        out_hbm.at[c, pl.ds(s * RPT, RPT)])

    return deg_kernel


def _make_propagate_kernel(n_phases, node_div):
    nph = NP // node_div
    rpt = nph // NTILES
    accr = nph + 64

    @functools.partial(
        pl.kernel,
        out_type=jax.ShapeDtypeStruct((n_phases, NP, 128), jnp.float32),
        mesh=_sc_mesh(),
        scratch_types=[
            pltpu.VMEM((CH, 128), jnp.int32),
            pltpu.VMEM((CH, 128), jnp.int32),
            pltpu.VMEM((128, 128), jnp.float32),
            pltpu.VMEM((128, 128), jnp.float32),
            pltpu.VMEM((128, 128), jnp.float32),
            pltpu.VMEM((128, 128), jnp.float32),
            pltpu.SemaphoreType.DMA,
            pltpu.SemaphoreType.DMA,
            pltpu.SemaphoreType.DMA,
            pltpu.SemaphoreType.DMA,
            pltpu.VMEM_SHARED((accr, 128), jnp.float32),
        ],
    )
    def prop_kernel(u_hbm, srcidx_hbm, dstidx_hbm, out_hbm,
                    sidx, didx, b00, b01, b10, b11,
                    gs0, gs1, ss0, ss1, acc):
        c = lax.axis_index("c")
        s = lax.axis_index("s")

        bufs = ((b00, b01), (b10, b11))
        gsem = (gs0, gs1)
        ssem = (ss0, ss1)
        NG = CH // 2

        def issue_gathers(g, p, k):
            for b in range(2):
                pltpu.async_copy(u_hbm.at[k].at[sidx.at[g * 2 + b]],
                                 bufs[p][b], gsem[p])

        def wait_gathers(p):
            for b in range(2):
                pltpu.make_async_copy(u_hbm.at[0].at[pl.ds(0, 128)],
                                      bufs[p][b], gsem[p]).wait()

        def issue_scatters(g, p):
            for b in range(2):
                pltpu.async_copy(bufs[p][b], acc.at[didx.at[g * 2 + b]],
                                 ssem[p], add=True)

        def wait_scatters(p):
            for b in range(2):
                pltpu.make_async_copy(bufs[p][b], acc.at[pl.ds(0, 128)],
                                      ssem[p]).wait()

        pltpu.sync_copy(srcidx_hbm.at[s], sidx)
        for k in range(n_phases):
            for sp in range(node_div // 2):
                pltpu.sync_copy(dstidx_hbm.at[sp, c, s], didx)

                @pl.loop(0, 128)
                def _(r):
                    for q in range(8):
                        b00[r, pl.ds(q * 16, 16)] = (
                            jnp.zeros((16,), jnp.float32))
                nfull, rem = divmod(rpt, 128)
                for z in range(nfull):
                    pltpu.sync_copy(
                        b00, acc.at[pl.ds(s * rpt + z * 128, 128)])
                if rem:
                    pltpu.sync_copy(
                        b00.at[pl.ds(0, rem)],
                        acc.at[pl.ds(s * rpt + nfull * 128, rem)])

                plsc.subcore_barrier()

                def grp(g, p):
                    wait_gathers(p)
                    issue_scatters(g, p)
                    wait_scatters(1 - p)
                    issue_gathers(g + 1, 1 - p, k)

                issue_gathers(0, 0, k)
                wait_gathers(0)
                issue_scatters(0, 0)
                issue_gathers(1, 1, k)

                @pl.loop(0, (NG - 2) // 2)
                def _(t):
                    grp(2 * t + 1, 1)
                    grp(2 * t + 2, 0)

                wait_gathers(1)
                issue_scatters(NG - 1, 1)
                wait_scatters(0)
                wait_scatters(1)
                plsc.subcore_barrier()
                pltpu.sync_copy(
                    acc.at[pl.ds(s * rpt, rpt)],
                    out_hbm.at[k, pl.ds((2 * sp + c) * nph + s * rpt, rpt)])

    return prop_kernel


@functools.lru_cache(maxsize=None)
def _get_degree_kernel():
    return _make_degree_kernel()


@functools.lru_cache(maxsize=None)
def _get_propagate_kernel(n_phases, node_div):
    return _make_propagate_kernel(n_phases, node_div)


def _degree(dstidx):
    return _get_degree_kernel()(dstidx)


def _prop_hidden(u, srcidx, dstidx_r, node_div):
    return _get_propagate_kernel(2, node_div)(u, srcidx, dstidx_r)


def _prop_out(u3p, srcidx, dstidx2):
    return _get_propagate_kernel(1, 2)(u3p.reshape(1, NP, 128), srcidx,
                                       dstidx2).reshape(NP, 128)


BR = 256
NB = NP // BR


def _dinv(deg_ref):
    d = deg_ref[:, 0:1] + deg_ref[:, 1:2] + 1.0
    return jax.lax.rsqrt(d)


def _cat2(u_ref):
    return jnp.concatenate([u_ref[0], u_ref[1]], axis=1)


def _halves(z):
    return jnp.stack([z[:, 0:128], z[:, 128:256]])


def _mm1_body(x_ref, w_ref, deg_ref, u_ref):
    dinv = _dinv(deg_ref)
    z = _dot(x_ref[...], w_ref[...]) * dinv
    u_ref[...] = _halves(z)


def _mm1(x_pad, W1, degT):
    return pl.pallas_call(
        _mm1_body,
        grid=(NB,),
        in_specs=[
            pl.BlockSpec((BR, D), lambda i: (i, 0)),
            pl.BlockSpec((D, HID), lambda i: (0, 0)),
            pl.BlockSpec((BR, 2), lambda i: (i, 0)),
        ],
        out_specs=pl.BlockSpec((2, BR, 128), lambda i: (0, i, 0)),
        out_shape=jax.ShapeDtypeStruct((2, NP, 128), jnp.float32),
    )(x_pad, W1, degT)


def _mm2_body(s_ref, u_ref, deg_ref, b_ref, w_ref, u2_ref, h_ref):
    dinv = _dinv(deg_ref)
    h1 = jnp.maximum(dinv * (_cat2(s_ref) + _cat2(u_ref)) + b_ref[...], 0.0)
    h_ref[...] = h1
    z = _dot(h1, w_ref[...]) * dinv
    u2_ref[...] = _halves(z)


def _mm2(s1, u1, degT, b2d, W):
    return pl.pallas_call(
        _mm2_body,
        grid=(NB,),
        in_specs=[
            pl.BlockSpec((2, BR, 128), lambda i: (0, i, 0)),
            pl.BlockSpec((2, BR, 128), lambda i: (0, i, 0)),
            pl.BlockSpec((BR, 2), lambda i: (i, 0)),
            pl.BlockSpec((1, HID), lambda i: (0, 0)),
            pl.BlockSpec((HID, HID), lambda i: (0, 0)),
        ],
        out_specs=[
            pl.BlockSpec((2, BR, 128), lambda i: (0, i, 0)),
            pl.BlockSpec((BR, HID), lambda i: (i, 0)),
        ],
        out_shape=[
            jax.ShapeDtypeStruct((2, NP, 128), jnp.float32),
            jax.ShapeDtypeStruct((NP, HID), jnp.float32),
        ],
    )(s1, u1, degT, b2d, W)


def _mm3_body(x_ref, h1_ref, s_ref, u_ref, deg_ref, b_ref, w_ref, u3_ref):
    dinv = _dinv(deg_ref)
    h2 = jnp.maximum(dinv * (_cat2(s_ref) + _cat2(u_ref)) + b_ref[...], 0.0)
    z = (_dot(x_ref[...], w_ref[0:D, :])
         + _dot(h1_ref[...], w_ref[D:D + HID, :])
         + _dot(h2, w_ref[D + HID:, :]))
    u = z * dinv
    u3_ref[...] = jnp.concatenate(
        [u, jnp.zeros((BR, 64), jnp.float32)], axis=1)


def _mm3(x_pad, h1, s2, u2, degT, b2d, Wo_pad):
    return pl.pallas_call(
        _mm3_body,
        grid=(NB,),
        in_specs=[
            pl.BlockSpec((BR, D), lambda i: (i, 0)),
            pl.BlockSpec((BR, HID), lambda i: (i, 0)),
            pl.BlockSpec((2, BR, 128), lambda i: (0, i, 0)),
            pl.BlockSpec((2, BR, 128), lambda i: (0, i, 0)),
            pl.BlockSpec((BR, 2), lambda i: (i, 0)),
            pl.BlockSpec((1, HID), lambda i: (0, 0)),
            pl.BlockSpec((D + 2 * HID, 64), lambda i: (0, 0)),
        ],
        out_specs=pl.BlockSpec((BR, 128), lambda i: (i, 0)),
        out_shape=jax.ShapeDtypeStruct((NP, 128), jnp.float32),
    )(x_pad, h1, s2, u2, degT, b2d, Wo_pad)


def _mm4_body(s_ref, u_ref, deg_ref, b_ref, o_ref):
    dinv = _dinv(deg_ref)
    cat = s_ref[:, 0:64] + u_ref[:, 0:64]
    o_ref[...] = jnp.maximum(dinv * cat + b_ref[...], 0.0)


def _mm4(s3, u3p, degT, bo2d):
    return pl.pallas_call(
        _mm4_body,
        grid=(NB,),
        in_specs=[
            pl.BlockSpec((BR, 128), lambda i: (i, 0)),
            pl.BlockSpec((BR, 128), lambda i: (i, 0)),
            pl.BlockSpec((BR, 2), lambda i: (i, 0)),
            pl.BlockSpec((1, 64), lambda i: (0, 0)),
        ],
        out_specs=pl.BlockSpec((BR, 64), lambda i: (i, 0)),
        out_shape=jax.ShapeDtypeStruct((NP, 64), jnp.float32),
    )(s3, u3p, degT, bo2d)


def kernel(x, edge_index, W1, b1, W2, b2, W_out, b_out):
    src = edge_index[0].astype(jnp.int32)
    dst = edge_index[1].astype(jnp.int32)

    src16 = src.reshape(NTILES, EPT)
    dst16 = dst.reshape(NTILES, EPT)
    pad_e = CH * 128 - EPT
    src_l = jnp.concatenate(
        [src16, jnp.zeros((NTILES, pad_e), jnp.int32)], axis=1
    ).reshape(NTILES, CH, 128)
    srcidx = src_l
    dstidx = jnp.concatenate(
        [dst16, jnp.full((NTILES, pad_e), TRASH, jnp.int32)], axis=1
    ).reshape(NTILES, CH, 128)
    dst_s = jnp.concatenate(
        [dst16, jnp.full((NTILES, pad_e), -1, jnp.int32)], axis=1
    ).reshape(NTILES, CH, 128)

    trash_spread = (jnp.arange(128, dtype=jnp.int32) % 64)[None, None, :]

    def route(node_div):
        nph = NP // node_div
        return jnp.stack([
            jnp.stack([
                jnp.where((dst_s >= q * nph) & (dst_s < (q + 1) * nph),
                          dst_s - q * nph, nph + trash_spread)
                for q in (2 * p, 2 * p + 1)], axis=0)
            for p in range(node_div // 2)])

    dstidx2 = route(2)

    x_pad = jnp.pad(x, ((0, NP - N), (0, 0)))
    Wo_pad = jnp.pad(W_out, ((0, 0), (0, 64 - NCLS)))
    bo_pad = jnp.pad(b_out, (0, 64 - NCLS)).reshape(1, 64)
    b1_2d = b1.reshape(1, HID)
    b2_2d = b2.reshape(1, HID)

    deg2 = _degree(dstidx)
    degT = deg2.T

    u1 = _mm1(x_pad, W1, degT)
    s1 = _prop_hidden(u1, srcidx, dstidx2, 2)
    u2, h1 = _mm2(s1, u1, degT, b1_2d, W2)
    s2 = _prop_hidden(u2, srcidx, dstidx2, 2)
    u3p = _mm3(x_pad, h1, s2, u2, degT, b2_2d, Wo_pad)
    s3 = _prop_out(u3p, srcidx, dstidx2)
    out = _mm4(s3, u3p, degT, bo_pad)
    return out[:N, :NCLS]
